# baseline (speedup 1.0000x reference)
import functools

import jax
import jax.numpy as jnp
from jax import lax
from jax.experimental import pallas as pl
from jax.experimental.pallas import tpu as pltpu
from jax.experimental.pallas import tpu_sc as plsc

_N = 10000
_E = 320000
_EPS = 1e-5

_BLK = 5000
_GRID = _N // _BLK

_NC = 2
_NS = 16
_NW = _NC * _NS
_C = 128
_NCHUNK = _E // _C
_BASE = _NCHUNK // _NW
_EXTRA = _NCHUNK % _NW
_RPT2 = 624
_LAST2 = _N - 15 * _RPT2
_NP1 = 10240
_RPT1 = _NP1 // _NS
_LAST1 = _RPT1


def _tile_copy(sid, src_at, dst_at, per, last):
    @pl.when(sid < 15)
    def _():
        pltpu.sync_copy(src_at(sid * per, per), dst_at(sid * per, per))

    @pl.when(sid == 15)
    def _():
        pltpu.sync_copy(src_at(15 * per, last), dst_at(15 * per, last))


_NPAIR = _BASE // 2


def _make_segsum(with_count):
    mesh = plsc.VectorSubcoreMesh(core_axis_name="c", subcore_axis_name="s")
    D = 128
    out_type = [jax.ShapeDtypeStruct((_NC, _N, D), jnp.float32)]
    scratch = [
        pltpu.VMEM((4, _C), jnp.int32),
        pltpu.VMEM((4, _C), jnp.int32),
        pltpu.VMEM((3, _C, D), jnp.float32),
        pltpu.VMEM_SHARED((_N, D), jnp.float32),
        pltpu.SemaphoreType.DMA((4,)),
        pltpu.SemaphoreType.DMA((2,)),
        pltpu.SemaphoreType.DMA((4,)),
        pltpu.SemaphoreType.DMA,
    ]
    if with_count:
        out_type.append(jax.ShapeDtypeStruct((_NC, _NP1), jnp.float32))
        scratch += [
            pltpu.VMEM((_C,), jnp.float32),
            pltpu.VMEM_SHARED((_NP1,), jnp.float32),
        ]

    def common(h_hbm, e_hbm, p_hbm, src_b, dst_b,
               rows, sem_i, sem_g, sem_s, sem_z, acc,
               ones_v=None, cacc=None):
        cid = lax.axis_index("c")
        sid = lax.axis_index("s")
        wid = sid * _NC + cid
        nloc = _BASE + jnp.where(wid < _EXTRA, 1, 0)

        def idx_start(c):
            off = (c * _NW + wid) * _C
            s4 = lax.rem(c, 4)
            pltpu.async_copy(e_hbm.at[0, pl.ds(off, _C)], src_b.at[s4],
                             sem_i.at[s4])
            pltpu.async_copy(e_hbm.at[1, pl.ds(off, _C)], dst_b.at[s4],
                             sem_i.at[s4])

        def idx_drain(c):
            off = (c * _NW + wid) * _C
            s4 = lax.rem(c, 4)
            pltpu.make_async_copy(e_hbm.at[0, pl.ds(off, _C)], src_b.at[s4],
                                  sem_i.at[s4]).wait()
            pltpu.make_async_copy(e_hbm.at[1, pl.ds(off, _C)], dst_b.at[s4],
                                  sem_i.at[s4]).wait()

        def gather_start(c):
            s4, s3, s2 = lax.rem(c, 4), lax.rem(c, 3), lax.rem(c, 2)
            pltpu.async_copy(h_hbm.at[src_b.at[s4]], rows.at[s3],
                             sem_g.at[s2])

        def gather_wait(c):
            s4, s3, s2 = lax.rem(c, 4), lax.rem(c, 3), lax.rem(c, 2)
            pltpu.make_async_copy(h_hbm.at[src_b.at[s4]], rows.at[s3],
                                  sem_g.at[s2]).wait()

        def scat_start(c):
            s4, s3 = lax.rem(c, 4), lax.rem(c, 3)
            pltpu.async_copy(rows.at[s3], acc.at[dst_b.at[s4]],
                             sem_s.at[s4], add=True)
            if ones_v is not None:
                pltpu.async_copy(ones_v, cacc.at[dst_b.at[s4]],
                                 sem_s.at[s4], add=True)

        def scat_drain(c):
            s4, s3 = lax.rem(c, 4), lax.rem(c, 3)
            pltpu.make_async_copy(rows.at[s3], acc.at[dst_b.at[s4]],
                                  sem_s.at[s4]).wait()
            if ones_v is not None:
                pltpu.make_async_copy(ones_v, cacc.at[dst_b.at[s4]],
                                      sem_s.at[s4]).wait()

        idx_start(jnp.int32(0))

        def zrow(i, carry):
            for k in range(8):
                rows[0, i, pl.ds(k * 16, 16)] = jnp.zeros((16,), jnp.float32)
            return carry

        lax.fori_loop(0, _C, zrow, 0)

        def zcopy(start):
            @pl.when(sid < 15)
            def _():
                base = sid * _RPT2
                for k in range(4):
                    start(rows.at[0], acc.at[pl.ds(base + k * _C, _C)])
                start(rows.at[0, pl.ds(0, _RPT2 - 4 * _C)],
                      acc.at[pl.ds(base + 4 * _C, _RPT2 - 4 * _C)])

            @pl.when(sid == 15)
            def _():
                base = 15 * _RPT2
                for k in range(5):
                    start(rows.at[0], acc.at[pl.ds(base + k * _C, _C)])

            if cacc is not None:
                base1 = sid * _RPT1
                for k in range(_RPT1 // _C):
                    start(rows.at[0, 0],
                          cacc.at[pl.ds(base1 + k * _C, _C)])

        zcopy(lambda s, d: pltpu.async_copy(s, d, sem_z))
        zcopy(lambda s, d: pltpu.make_async_copy(s, d, sem_z).wait())
        plsc.subcore_barrier()

        def step(c, carry):
            @pl.when(c >= 3)
            def _():
                scat_drain(c - 3)

            idx_drain(c)
            gather_start(c)

            @pl.when(c > 0)
            def _():
                gather_wait(c - 1)
                scat_start(c - 1)

            @pl.when(c + 1 < nloc)
            def _():
                idx_start(c + 1)
            return carry

        lax.fori_loop(0, _BASE, step, 0, unroll=2)

        scat_drain(jnp.int32(_BASE - 3))
        gather_wait(jnp.int32(_BASE - 1))
        scat_start(jnp.int32(_BASE - 1))

        @pl.when(wid < _EXTRA)
        def _():
            c = jnp.int32(_BASE)
            idx_drain(c)
            gather_start(c)
            gather_wait(c)
            scat_start(c)

        scat_drain(jnp.int32(_BASE - 2))
        scat_drain(jnp.int32(_BASE - 1))

        @pl.when(wid < _EXTRA)
        def _():
            scat_drain(jnp.int32(_BASE))

        plsc.subcore_barrier()
        _tile_copy(sid, lambda o, n: acc.at[pl.ds(o, n)],
                   lambda o, n: p_hbm.at[cid].at[pl.ds(o, n)], _RPT2, _LAST2)
        return cid, sid

    if with_count:
        def body(h_hbm, e_hbm, p_hbm, c_hbm,
                 src_b, dst_b, rows, acc, sem_i, sem_g, sem_s, sem_z,
                 ones_v, cacc):
            for i in range(_C // 16):
                ones_v[pl.ds(i * 16, 16)] = jnp.ones((16,), jnp.float32)

            cid, sid = common(h_hbm, e_hbm, p_hbm,
                              src_b, dst_b, rows, sem_i, sem_g, sem_s,
                              sem_z, acc, ones_v=ones_v, cacc=cacc)

            _tile_copy(sid, lambda o, n: cacc.at[pl.ds(o, n)],
                       lambda o, n: c_hbm.at[cid].at[pl.ds(o, n)],
                       _RPT1, _LAST1)
    else:
        def body(h_hbm, e_hbm, p_hbm,
                 src_b, dst_b, rows, acc, sem_i, sem_g, sem_s, sem_z):
            common(h_hbm, e_hbm, p_hbm,
                   src_b, dst_b, rows, sem_i, sem_g, sem_s, sem_z, acc)

    return pl.kernel(body, out_type=out_type, mesh=mesh, scratch_types=scratch)


_segsum_count = _make_segsum(True)
_segsum_128 = _make_segsum(False)


def _full(shape):
    return pl.BlockSpec(shape, lambda i: tuple(0 for _ in shape))


def _proj_body(x_ref, wl_ref, wr_ref, b_ref, hl_ref, hr_ref):
    x = x_ref[...]
    hl_ref[...] = jnp.dot(x, wl_ref[...], preferred_element_type=jnp.float32)
    hr_ref[...] = (jnp.dot(x, wr_ref[...], preferred_element_type=jnp.float32)
                   + b_ref[...])


def _proj(x, Wl, Wr, b, Do):
    return pl.pallas_call(
        _proj_body,
        grid=(_GRID,),
        in_specs=[
            pl.BlockSpec((_BLK, 128), lambda i: (i, 0)),
            _full((128, Do)),
            _full((128, Do)),
            _full((1, Do)),
        ],
        out_specs=[pl.BlockSpec((_BLK, Do), lambda i: (i, 0))] * 2,
        out_shape=[jax.ShapeDtypeStruct((_N, Do), jnp.float32)] * 2,
    )(x, Wl, Wr, b.reshape(1, Do))


def _mid_body(emit_h, p_ref, c_ref, hr, g, be, wl, wr, b,
              hl_ref, hro_ref, t_sc, st_sc):
    ph = pl.program_id(0)
    i = pl.program_id(1)

    @pl.when(ph == 0)
    def _():
        cnt = c_ref[0] + c_ref[1]
        inv = 1.0 / jnp.maximum(cnt, 1.0)
        t = (p_ref[0] + p_ref[1]) * inv + hr[...]
        t_sc[i] = t
        s = jnp.concatenate(
            [jnp.sum(t, 0, keepdims=True), jnp.sum(t * t, 0, keepdims=True)],
            0)

        @pl.when(i == 0)
        def _():
            st_sc[...] = s

        @pl.when(i != 0)
        def _():
            st_sc[...] += s

    @pl.when(ph == 1)
    def _():
        mu = st_sc[0:1, :] * (1.0 / _N)
        var = st_sc[1:2, :] * (1.0 / _N) - mu * mu
        h = jnp.maximum(
            (t_sc[i] - mu) * lax.rsqrt(var + _EPS) * g[...] + be[...], 0.0)
        if emit_h:
            hl_ref[...] = h
        else:
            hl_ref[...] = jnp.dot(h, wl[...],
                                  preferred_element_type=jnp.float32)
        hro_ref[...] = (jnp.dot(h, wr[...],
                                preferred_element_type=jnp.float32)
                        + b[...])


def _mid(p, c, hr, g, be, Wl, Wr, b, Do, emit_h=False):
    hl_w = 128 if emit_h else Do
    return pl.pallas_call(
        functools.partial(_mid_body, emit_h),
        grid=(2, _GRID),
        in_specs=[
            pl.BlockSpec((2, _BLK, 128), lambda ph, i: (0, i * (1 - ph), 0)),
            pl.BlockSpec((2, _BLK, 1), lambda ph, i: (0, i * (1 - ph), 0)),
            pl.BlockSpec((_BLK, 128), lambda ph, i: (i * (1 - ph), 0)),
            pl.BlockSpec((1, 128), lambda ph, i: (0, 0)),
            pl.BlockSpec((1, 128), lambda ph, i: (0, 0)),
            pl.BlockSpec((128, Do), lambda ph, i: (0, 0)),
            pl.BlockSpec((128, Do), lambda ph, i: (0, 0)),
            pl.BlockSpec((1, Do), lambda ph, i: (0, 0)),
        ],
        out_specs=[
            pl.BlockSpec((_BLK, hl_w), lambda ph, i: (i * ph, 0)),
            pl.BlockSpec((_BLK, Do), lambda ph, i: (i * ph, 0)),
        ],
        out_shape=[
            jax.ShapeDtypeStruct((_N, hl_w), jnp.float32),
            jax.ShapeDtypeStruct((_N, Do), jnp.float32),
        ],
        scratch_shapes=[
            pltpu.VMEM((_GRID, _BLK, 128), jnp.float32),
            pltpu.VMEM((2, 128), jnp.float32),
        ],
    )(p, c, hr, g.reshape(1, 128), be.reshape(1, 128),
      Wl, Wr, b.reshape(1, Do))


def _final_body(p_ref, c_ref, hr_ref, wl_ref, o_ref):
    cnt = c_ref[0] + c_ref[1]
    inv = 1.0 / jnp.maximum(cnt, 1.0)
    mean = (p_ref[0] + p_ref[1]) * inv
    t = (jnp.dot(mean, wl_ref[...], preferred_element_type=jnp.float32)
         + hr_ref[...])
    m = jnp.max(t, -1, keepdims=True)
    lse = jnp.log(jnp.sum(jnp.exp(t - m), -1, keepdims=True)) + m
    o_ref[...] = t - lse


def _final(p, c, hr, Wl):
    return pl.pallas_call(
        _final_body,
        grid=(_GRID,),
        in_specs=[
            pl.BlockSpec((2, _BLK, 128), lambda i: (0, i, 0)),
            pl.BlockSpec((2, _BLK, 1), lambda i: (0, i, 0)),
            pl.BlockSpec((_BLK, 64), lambda i: (i, 0)),
            _full((128, 64)),
        ],
        out_specs=pl.BlockSpec((_BLK, 64), lambda i: (i, 0)),
        out_shape=jax.ShapeDtypeStruct((_N, 64), jnp.float32),
    )(p, c, hr, Wl)


def kernel(x, edge_index, Wl0, bl0, Wr0, g0, be0,
           Wl1, bl1, Wr1, g1, be1, Wl2, bl2, Wr2):
    hl0, hr0 = _proj(x, Wl0, Wr0, bl0, 128)
    p0, cnt = _segsum_count(hl0, edge_index)
    c = cnt.reshape(_NC, _NP1, 1)

    hl1, hr1 = _mid(p0, c, hr0, g0, be0, Wl1, Wr1, bl1, 128)
    [p1] = _segsum_128(hl1, edge_index)

    h2, hr2 = _mid(p1, c, hr1, g1, be1, Wl2, Wr2, bl2, 64, emit_h=True)
    [p2] = _segsum_128(h2, edge_index)
    return _final(p2, c, hr2, Wl2)


# speedup vs baseline: 1.0155x; 1.0155x over previous
import functools

import jax
import jax.numpy as jnp
from jax import lax
from jax.experimental import pallas as pl
from jax.experimental.pallas import tpu as pltpu
from jax.experimental.pallas import tpu_sc as plsc

_N = 10000
_E = 320000
_EPS = 1e-5

_BLK = 2000
_GRID = _N // _BLK

_NC = 2
_NS = 16
_NW = _NC * _NS
_C = 128
_NCHUNK = _E // _C
_BASE = _NCHUNK // _NW
_EXTRA = _NCHUNK % _NW
_RPT2 = 624
_LAST2 = _N - 15 * _RPT2
_NP1 = 10240
_RPT1 = _NP1 // _NS
_LAST1 = _RPT1


def _tile_copy(sid, src_at, dst_at, per, last):
    @pl.when(sid < 15)
    def _():
        pltpu.sync_copy(src_at(sid * per, per), dst_at(sid * per, per))

    @pl.when(sid == 15)
    def _():
        pltpu.sync_copy(src_at(15 * per, last), dst_at(15 * per, last))


_NPAIR = _BASE // 2


def _make_segsum(with_count):
    mesh = plsc.VectorSubcoreMesh(core_axis_name="c", subcore_axis_name="s")
    D = 128
    out_type = [jax.ShapeDtypeStruct((_NC, _N, D), jnp.float32)]
    scratch = [
        pltpu.VMEM((4, _C), jnp.int32),
        pltpu.VMEM((4, _C), jnp.int32),
        pltpu.VMEM((3, _C, D), jnp.float32),
        pltpu.VMEM_SHARED((_N, D), jnp.float32),
        pltpu.SemaphoreType.DMA((4,)),
        pltpu.SemaphoreType.DMA((2,)),
        pltpu.SemaphoreType.DMA((4,)),
        pltpu.SemaphoreType.DMA,
    ]
    if with_count:
        out_type.append(jax.ShapeDtypeStruct((_NC, _NP1), jnp.float32))
        scratch += [
            pltpu.VMEM((_C,), jnp.float32),
            pltpu.VMEM_SHARED((_NP1,), jnp.float32),
        ]

    def common(h_hbm, e_hbm, p_hbm, src_b, dst_b,
               rows, sem_i, sem_g, sem_s, sem_z, acc,
               ones_v=None, cacc=None):
        cid = lax.axis_index("c")
        sid = lax.axis_index("s")
        wid = sid * _NC + cid
        nloc = _BASE + jnp.where(wid < _EXTRA, 1, 0)

        def idx_start(c):
            off = (c * _NW + wid) * _C
            s4 = lax.rem(c, 4)
            pltpu.async_copy(e_hbm.at[0, pl.ds(off, _C)], src_b.at[s4],
                             sem_i.at[s4])
            pltpu.async_copy(e_hbm.at[1, pl.ds(off, _C)], dst_b.at[s4],
                             sem_i.at[s4])

        def idx_drain(c):
            off = (c * _NW + wid) * _C
            s4 = lax.rem(c, 4)
            pltpu.make_async_copy(e_hbm.at[0, pl.ds(off, _C)], src_b.at[s4],
                                  sem_i.at[s4]).wait()
            pltpu.make_async_copy(e_hbm.at[1, pl.ds(off, _C)], dst_b.at[s4],
                                  sem_i.at[s4]).wait()

        def gather_start(c):
            s4, s3, s2 = lax.rem(c, 4), lax.rem(c, 3), lax.rem(c, 2)
            pltpu.async_copy(h_hbm.at[src_b.at[s4]], rows.at[s3],
                             sem_g.at[s2])

        def gather_wait(c):
            s4, s3, s2 = lax.rem(c, 4), lax.rem(c, 3), lax.rem(c, 2)
            pltpu.make_async_copy(h_hbm.at[src_b.at[s4]], rows.at[s3],
                                  sem_g.at[s2]).wait()

        def scat_start(c):
            s4, s3 = lax.rem(c, 4), lax.rem(c, 3)
            pltpu.async_copy(rows.at[s3], acc.at[dst_b.at[s4]],
                             sem_s.at[s4], add=True)
            if ones_v is not None:
                pltpu.async_copy(ones_v, cacc.at[dst_b.at[s4]],
                                 sem_s.at[s4], add=True)

        def scat_drain(c):
            s4, s3 = lax.rem(c, 4), lax.rem(c, 3)
            pltpu.make_async_copy(rows.at[s3], acc.at[dst_b.at[s4]],
                                  sem_s.at[s4]).wait()
            if ones_v is not None:
                pltpu.make_async_copy(ones_v, cacc.at[dst_b.at[s4]],
                                      sem_s.at[s4]).wait()

        idx_start(jnp.int32(0))

        def zrow(i, carry):
            for k in range(8):
                rows[2, i, pl.ds(k * 16, 16)] = jnp.zeros((16,), jnp.float32)
            return carry

        lax.fori_loop(0, _C, zrow, 0)

        def zcopy(start):
            @pl.when(sid < 15)
            def _():
                base = sid * _RPT2
                for k in range(4):
                    start(rows.at[2], acc.at[pl.ds(base + k * _C, _C)])
                start(rows.at[2, pl.ds(0, _RPT2 - 4 * _C)],
                      acc.at[pl.ds(base + 4 * _C, _RPT2 - 4 * _C)])

            @pl.when(sid == 15)
            def _():
                base = 15 * _RPT2
                for k in range(5):
                    start(rows.at[2], acc.at[pl.ds(base + k * _C, _C)])

            if cacc is not None:
                base1 = sid * _RPT1
                for k in range(_RPT1 // _C):
                    start(rows.at[2, 0],
                          cacc.at[pl.ds(base1 + k * _C, _C)])

        zcopy(lambda s, d: pltpu.async_copy(s, d, sem_z))
        idx_drain(jnp.int32(0))
        gather_start(jnp.int32(0))
        idx_start(jnp.int32(1))
        zcopy(lambda s, d: pltpu.make_async_copy(s, d, sem_z).wait())
        plsc.subcore_barrier()

        def step(c, carry):
            @pl.when(c >= 3)
            def _():
                scat_drain(c - 3)

            idx_drain(c)
            gather_start(c)

            @pl.when(c > 0)
            def _():
                gather_wait(c - 1)
                scat_start(c - 1)

            @pl.when(c + 1 < nloc)
            def _():
                idx_start(c + 1)
            return carry

        lax.fori_loop(1, _BASE, step, 0, unroll=2)

        scat_drain(jnp.int32(_BASE - 3))
        gather_wait(jnp.int32(_BASE - 1))
        scat_start(jnp.int32(_BASE - 1))

        @pl.when(wid < _EXTRA)
        def _():
            c = jnp.int32(_BASE)
            idx_drain(c)
            gather_start(c)
            gather_wait(c)
            scat_start(c)

        scat_drain(jnp.int32(_BASE - 2))
        scat_drain(jnp.int32(_BASE - 1))

        @pl.when(wid < _EXTRA)
        def _():
            scat_drain(jnp.int32(_BASE))

        plsc.subcore_barrier()
        _tile_copy(sid, lambda o, n: acc.at[pl.ds(o, n)],
                   lambda o, n: p_hbm.at[cid].at[pl.ds(o, n)], _RPT2, _LAST2)
        return cid, sid

    if with_count:
        def body(h_hbm, e_hbm, p_hbm, c_hbm,
                 src_b, dst_b, rows, acc, sem_i, sem_g, sem_s, sem_z,
                 ones_v, cacc):
            for i in range(_C // 16):
                ones_v[pl.ds(i * 16, 16)] = jnp.ones((16,), jnp.float32)

            cid, sid = common(h_hbm, e_hbm, p_hbm,
                              src_b, dst_b, rows, sem_i, sem_g, sem_s,
                              sem_z, acc, ones_v=ones_v, cacc=cacc)

            _tile_copy(sid, lambda o, n: cacc.at[pl.ds(o, n)],
                       lambda o, n: c_hbm.at[cid].at[pl.ds(o, n)],
                       _RPT1, _LAST1)
    else:
        def body(h_hbm, e_hbm, p_hbm,
                 src_b, dst_b, rows, acc, sem_i, sem_g, sem_s, sem_z):
            common(h_hbm, e_hbm, p_hbm,
                   src_b, dst_b, rows, sem_i, sem_g, sem_s, sem_z, acc)

    return pl.kernel(body, out_type=out_type, mesh=mesh, scratch_types=scratch)


_segsum_count = _make_segsum(True)
_segsum_128 = _make_segsum(False)


def _full(shape):
    return pl.BlockSpec(shape, lambda i: tuple(0 for _ in shape))


def _proj_body(x_ref, wl_ref, wr_ref, b_ref, hl_ref, hr_ref):
    x = x_ref[...]
    hl_ref[...] = jnp.dot(x, wl_ref[...], preferred_element_type=jnp.float32)
    hr_ref[...] = (jnp.dot(x, wr_ref[...], preferred_element_type=jnp.float32)
                   + b_ref[...])


def _proj(x, Wl, Wr, b, Do):
    return pl.pallas_call(
        _proj_body,
        grid=(_GRID,),
        in_specs=[
            pl.BlockSpec((_BLK, 128), lambda i: (i, 0)),
            _full((128, Do)),
            _full((128, Do)),
            _full((1, Do)),
        ],
        out_specs=[pl.BlockSpec((_BLK, Do), lambda i: (i, 0))] * 2,
        out_shape=[jax.ShapeDtypeStruct((_N, Do), jnp.float32)] * 2,
    )(x, Wl, Wr, b.reshape(1, Do))


def _mid_body(emit_h, p_ref, c_ref, hr, g, be, wl, wr, b,
              hl_ref, hro_ref, t_sc, st_sc):
    ph = pl.program_id(0)
    i = pl.program_id(1)

    @pl.when(ph == 0)
    def _():
        cnt = c_ref[0] + c_ref[1]
        inv = 1.0 / jnp.maximum(cnt, 1.0)
        t = (p_ref[0] + p_ref[1]) * inv + hr[...]
        t_sc[i] = t
        s = jnp.concatenate(
            [jnp.sum(t, 0, keepdims=True), jnp.sum(t * t, 0, keepdims=True)],
            0)

        @pl.when(i == 0)
        def _():
            st_sc[...] = s

        @pl.when(i != 0)
        def _():
            st_sc[...] += s

    @pl.when(ph == 1)
    def _():
        mu = st_sc[0:1, :] * (1.0 / _N)
        var = st_sc[1:2, :] * (1.0 / _N) - mu * mu
        h = jnp.maximum(
            (t_sc[i] - mu) * lax.rsqrt(var + _EPS) * g[...] + be[...], 0.0)
        if emit_h:
            hl_ref[...] = h
        else:
            hl_ref[...] = jnp.dot(h, wl[...],
                                  preferred_element_type=jnp.float32)
        hro_ref[...] = (jnp.dot(h, wr[...],
                                preferred_element_type=jnp.float32)
                        + b[...])


def _mid(p, c, hr, g, be, Wl, Wr, b, Do, emit_h=False):
    hl_w = 128 if emit_h else Do
    return pl.pallas_call(
        functools.partial(_mid_body, emit_h),
        grid=(2, _GRID),
        in_specs=[
            pl.BlockSpec((2, _BLK, 128), lambda ph, i: (0, i * (1 - ph), 0)),
            pl.BlockSpec((2, _BLK, 1), lambda ph, i: (0, i * (1 - ph), 0)),
            pl.BlockSpec((_BLK, 128), lambda ph, i: (i * (1 - ph), 0)),
            pl.BlockSpec((1, 128), lambda ph, i: (0, 0)),
            pl.BlockSpec((1, 128), lambda ph, i: (0, 0)),
            pl.BlockSpec((128, Do), lambda ph, i: (0, 0)),
            pl.BlockSpec((128, Do), lambda ph, i: (0, 0)),
            pl.BlockSpec((1, Do), lambda ph, i: (0, 0)),
        ],
        out_specs=[
            pl.BlockSpec((_BLK, hl_w), lambda ph, i: (i * ph, 0)),
            pl.BlockSpec((_BLK, Do), lambda ph, i: (i * ph, 0)),
        ],
        out_shape=[
            jax.ShapeDtypeStruct((_N, hl_w), jnp.float32),
            jax.ShapeDtypeStruct((_N, Do), jnp.float32),
        ],
        scratch_shapes=[
            pltpu.VMEM((_GRID, _BLK, 128), jnp.float32),
            pltpu.VMEM((2, 128), jnp.float32),
        ],
    )(p, c, hr, g.reshape(1, 128), be.reshape(1, 128),
      Wl, Wr, b.reshape(1, Do))


def _final_body(p_ref, c_ref, hr_ref, wl_ref, o_ref):
    cnt = c_ref[0] + c_ref[1]
    inv = 1.0 / jnp.maximum(cnt, 1.0)
    mean = (p_ref[0] + p_ref[1]) * inv
    t = (jnp.dot(mean, wl_ref[...], preferred_element_type=jnp.float32)
         + hr_ref[...])
    m = jnp.max(t, -1, keepdims=True)
    lse = jnp.log(jnp.sum(jnp.exp(t - m), -1, keepdims=True)) + m
    o_ref[...] = t - lse


def _final(p, c, hr, Wl):
    return pl.pallas_call(
        _final_body,
        grid=(_GRID,),
        in_specs=[
            pl.BlockSpec((2, _BLK, 128), lambda i: (0, i, 0)),
            pl.BlockSpec((2, _BLK, 1), lambda i: (0, i, 0)),
            pl.BlockSpec((_BLK, 64), lambda i: (i, 0)),
            _full((128, 64)),
        ],
        out_specs=pl.BlockSpec((_BLK, 64), lambda i: (i, 0)),
        out_shape=jax.ShapeDtypeStruct((_N, 64), jnp.float32),
    )(p, c, hr, Wl)


def kernel(x, edge_index, Wl0, bl0, Wr0, g0, be0,
           Wl1, bl1, Wr1, g1, be1, Wl2, bl2, Wr2):
    hl0, hr0 = _proj(x, Wl0, Wr0, bl0, 128)
    p0, cnt = _segsum_count(hl0, edge_index)
    c = cnt.reshape(_NC, _NP1, 1)

    hl1, hr1 = _mid(p0, c, hr0, g0, be0, Wl1, Wr1, bl1, 128)
    [p1] = _segsum_128(hl1, edge_index)

    h2, hr2 = _mid(p1, c, hr1, g1, be1, Wl2, Wr2, bl2, 64, emit_h=True)
    [p2] = _segsum_128(h2, edge_index)
    return _final(p2, c, hr2, Wl2)
